# Initial kernel scaffold; baseline (speedup 1.0000x reference)
import jax
import jax.numpy as jnp
from jax.experimental import pallas as pl


def kernel(fake_images, real_images):
    raise NotImplementedError("write your pallas kernel here")


# speedup vs baseline: 43.9054x; 43.9054x over previous
import functools

import jax
import jax.numpy as jnp
from jax import lax
from jax.experimental import pallas as pl
from jax.experimental.pallas import tpu as pltpu
from jax.experimental.pallas import tpu_sc as plsc

B, C, H, W = 16, 3, 512, 512
BINS = 64
ROWS = B * C
ROW_ELEMS = H * W
TOTAL = ROWS * ROW_ELEMS

NC, NS, L = 2, 16, 16
ROWS_PER_SUB = ROWS // NS
SUB_ELEMS = ROWS_PER_SUB * ROW_ELEMS
CHUNK = 32768
NCHUNKS = SUB_ELEMS // CHUNK
CHUNKS_PER_ROW = ROW_ELEMS // CHUNK
VECS = CHUNK // L
TBL = ROWS_PER_SUB * BINS


def _histograms_sc(fake_flat, real_flat):
  mesh = plsc.VectorSubcoreMesh(
      core_axis_name="c", subcore_axis_name="s", num_cores=NC,
      num_subcores=NS)

  @functools.partial(
      pl.kernel,
      out_type=jax.ShapeDtypeStruct((NC * ROWS * BINS,), jnp.float32),
      mesh=mesh,
      compiler_params=pltpu.CompilerParams(needs_layout_passes=False),
      scratch_types=[
          pltpu.VMEM((CHUNK,), jnp.float32),
          pltpu.VMEM((CHUNK,), jnp.float32),
          pltpu.VMEM((L * TBL,), jnp.float32),
          pltpu.VMEM((TBL,), jnp.float32),
          pltpu.SemaphoreType.DMA,
          pltpu.SemaphoreType.DMA,
      ],
  )
  def hist_kernel(fake_hbm, real_hbm, out_hbm, buf0, buf1, table, acc,
                  sem0, sem1):
    c = lax.axis_index("c")
    s = lax.axis_index("s")

    def run(img_hbm, c_lit):
      zeros = jnp.zeros((L,), jnp.float32)

      def zbody(j, carry):
        table[pl.ds(j * L, L)] = zeros
        return carry

      lax.fori_loop(0, TBL, zbody, 0)

      base = s * SUB_ELEMS
      bufs = (buf0, buf1)
      sems = (sem0, sem1)
      lane_base = jnp.arange(L, dtype=jnp.int32) * TBL
      ones = jnp.ones((L,), jnp.float32)

      pending = [None, None]
      pending[0] = pltpu.async_copy(
          img_hbm.at[pl.ds(base, CHUNK)], bufs[0], sems[0])
      for k in range(NCHUNKS):
        nb = k & 1
        if k + 1 < NCHUNKS:
          pending[1 - nb] = pltpu.async_copy(
              img_hbm.at[pl.ds(base + (k + 1) * CHUNK, CHUNK)],
              bufs[1 - nb], sems[1 - nb])
        pending[nb].wait()
        buf = bufs[nb]
        row = k // CHUNKS_PER_ROW
        cvec = lane_base + jnp.int32(row * BINS)

        def vbody(i, carry, buf=buf, cvec=cvec):
          x = buf[pl.ds(i * L, L)]
          f = x * jnp.float32(BINS)
          f = jnp.minimum(jnp.maximum(f, 0.0), jnp.float32(BINS - 1))
          addr = f.astype(jnp.int32) + cvec
          plsc.addupdate_scatter(table, [addr], ones)
          return carry

        lax.fori_loop(0, VECS, vbody, 0, unroll=8)

      for j in range(TBL // L):
        v = table[pl.ds(j * L, L)]
        for l in range(1, L):
          v = v + table[pl.ds(l * TBL + j * L, L)]
        acc[pl.ds(j * L, L)] = v

      off = c_lit * (ROWS * BINS) + s * TBL
      pltpu.sync_copy(acc, out_hbm.at[pl.ds(off, TBL)])

    @pl.when(c == 0)
    def _():
      run(fake_hbm, 0)

    @pl.when(c == 1)
    def _():
      run(real_hbm, 1)

  return hist_kernel(fake_flat, real_flat)


def _loss_body(h_ref, o_ref):
  h = h_ref[...]
  ssum = jnp.clip(jnp.sum(h, axis=1, keepdims=True), 1e-8, None)
  n = h / ssum
  d = jnp.abs(n[:ROWS] - n[ROWS:])
  o_ref[0, 0] = jnp.sum(d) / jnp.float32(ROWS * BINS)


def kernel(fake_images, real_images):
  fake_flat = fake_images.reshape(TOTAL)
  real_flat = real_images.reshape(TOTAL)
  hists = _histograms_sc(fake_flat, real_flat).reshape(NC * ROWS, BINS)
  loss = pl.pallas_call(
      _loss_body,
      out_shape=jax.ShapeDtypeStruct((1, 1), jnp.float32),
      out_specs=pl.BlockSpec(memory_space=pltpu.SMEM),
  )(hists)
  return loss[0, 0]


# speedup vs baseline: 144.7105x; 3.2960x over previous
import functools

import jax
import jax.numpy as jnp
from jax import lax
from jax.experimental import pallas as pl
from jax.experimental.pallas import tpu as pltpu
from jax.experimental.pallas import tpu_sc as plsc

B, C, H, W = 16, 3, 512, 512
BINS = 64
ROWS = B * C
ROW_ELEMS = H * W
TOTAL = ROWS * ROW_ELEMS

NC, NS, L = 2, 16, 16
ROWS_PER_SUB = ROWS // NS
SUB_ELEMS = ROWS_PER_SUB * ROW_ELEMS
CHUNK = 32768
NCHUNKS = SUB_ELEMS // CHUNK
CHUNKS_PER_ROW = ROW_ELEMS // CHUNK
VECS = CHUNK // L
TBL = ROWS_PER_SUB * BINS


def _histograms_sc(fake_flat, real_flat):
  mesh = plsc.VectorSubcoreMesh(
      core_axis_name="c", subcore_axis_name="s", num_cores=NC,
      num_subcores=NS)

  @functools.partial(
      pl.kernel,
      out_type=jax.ShapeDtypeStruct((NC * ROWS * BINS,), jnp.float32),
      mesh=mesh,
      compiler_params=pltpu.CompilerParams(needs_layout_passes=False),
      scratch_types=[
          pltpu.VMEM((CHUNK,), jnp.float32),
          pltpu.VMEM((CHUNK,), jnp.float32),
          pltpu.VMEM((L * TBL,), jnp.float32),
          pltpu.VMEM((TBL,), jnp.float32),
          pltpu.SemaphoreType.DMA,
          pltpu.SemaphoreType.DMA,
      ],
  )
  def hist_kernel(fake_hbm, real_hbm, out_hbm, buf0, buf1, table, acc,
                  sem0, sem1):
    c = lax.axis_index("c")
    s = lax.axis_index("s")

    def run(img_hbm, c_lit):
      zeros = jnp.zeros((L,), jnp.float32)

      @plsc.parallel_loop(0, TBL, unroll=8)
      def _(j):
        table[pl.ds(j * L, L)] = zeros

      base = s * SUB_ELEMS
      bufs = (buf0, buf1)
      sems = (sem0, sem1)
      lane_base = jnp.arange(L, dtype=jnp.int32) * TBL
      ones = jnp.ones((L,), jnp.float32)

      pending = [None, None]
      pending[0] = pltpu.async_copy(
          img_hbm.at[pl.ds(base, CHUNK)], bufs[0], sems[0])
      for k in range(NCHUNKS):
        nb = k & 1
        if k + 1 < NCHUNKS:
          pending[1 - nb] = pltpu.async_copy(
              img_hbm.at[pl.ds(base + (k + 1) * CHUNK, CHUNK)],
              bufs[1 - nb], sems[1 - nb])
        pending[nb].wait()
        buf = bufs[nb]
        row = k // CHUNKS_PER_ROW
        cvec = lane_base + jnp.int32(row * BINS)

        @plsc.parallel_loop(0, VECS, unroll=8)
        def _(i, buf=buf, cvec=cvec):
          x = buf[pl.ds(i * L, L)]
          f = x * jnp.float32(BINS)
          f = jnp.minimum(jnp.maximum(f, 0.0), jnp.float32(BINS - 1))
          addr = f.astype(jnp.int32) + cvec
          plsc.addupdate_scatter(table, [addr], ones)

      for j in range(TBL // L):
        v = table[pl.ds(j * L, L)]
        for l in range(1, L):
          v = v + table[pl.ds(l * TBL + j * L, L)]
        acc[pl.ds(j * L, L)] = v

      off = c_lit * (ROWS * BINS) + s * TBL
      pltpu.sync_copy(acc, out_hbm.at[pl.ds(off, TBL)])

    @pl.when(c == 0)
    def _():
      run(fake_hbm, 0)

    @pl.when(c == 1)
    def _():
      run(real_hbm, 1)

  return hist_kernel(fake_flat, real_flat)


def _loss_body(h_ref, o_ref):
  h = h_ref[...]
  ssum = jnp.clip(jnp.sum(h, axis=1, keepdims=True), 1e-8, None)
  n = h / ssum
  d = jnp.abs(n[:ROWS] - n[ROWS:])
  o_ref[0, 0] = jnp.sum(d) / jnp.float32(ROWS * BINS)


def kernel(fake_images, real_images):
  fake_flat = fake_images.reshape(TOTAL)
  real_flat = real_images.reshape(TOTAL)
  hists = _histograms_sc(fake_flat, real_flat).reshape(NC * ROWS, BINS)
  loss = pl.pallas_call(
      _loss_body,
      out_shape=jax.ShapeDtypeStruct((1, 1), jnp.float32),
      out_specs=pl.BlockSpec(memory_space=pltpu.SMEM),
  )(hists)
  return loss[0, 0]


# speedup vs baseline: 208.9725x; 1.4441x over previous
import functools

import jax
import jax.numpy as jnp
from jax import lax
from jax.experimental import pallas as pl
from jax.experimental.pallas import tpu as pltpu
from jax.experimental.pallas import tpu_sc as plsc

B, C, H, W = 16, 3, 512, 512
BINS = 64
ROWS = B * C

NC, NS, L = 2, 16, 16
CH_ROWS = 64
CHUNK = CH_ROWS * W
CHUNKS_PER_CH = H // CH_ROWS
VECS_PER_ROW = W // L
TBL = C * BINS


def _histograms_sc(fake_images, real_images):
  mesh = plsc.VectorSubcoreMesh(
      core_axis_name="c", subcore_axis_name="s", num_cores=NC,
      num_subcores=NS)

  @functools.partial(
      pl.kernel,
      out_type=jax.ShapeDtypeStruct((NC * NS, TBL), jnp.float32),
      mesh=mesh,
      compiler_params=pltpu.CompilerParams(
          needs_layout_passes=False, use_tc_tiling_on_sc=True),
      scratch_types=[
          pltpu.VMEM((CH_ROWS, W), jnp.float32),
          pltpu.VMEM((CH_ROWS, W), jnp.float32),
          pltpu.VMEM((L * TBL,), jnp.float32),
          pltpu.VMEM((TBL,), jnp.float32),
          pltpu.SemaphoreType.DMA,
          pltpu.SemaphoreType.DMA,
      ],
  )
  def hist_kernel(fake_hbm, real_hbm, out_hbm, buf0, buf1, table, acc,
                  sem0, sem1):
    c = lax.axis_index("c")
    s = lax.axis_index("s")
    bufs = (buf0, buf1)
    sems = (sem0, sem1)
    lane_base = jnp.arange(L, dtype=jnp.int32) * TBL
    ones = jnp.ones((L,), jnp.float32)
    zeros = jnp.zeros((L,), jnp.float32)
    nchunks = C * CHUNKS_PER_CH

    @plsc.parallel_loop(0, TBL, unroll=8)
    def _(j):
      table[pl.ds(j * L, L)] = zeros

    def run(img_hbm):

      def start(k, nb):
        ch = lax.shift_right_logical(k, 3)
        blk = lax.bitwise_and(k, CHUNKS_PER_CH - 1)
        pltpu.async_copy(
            img_hbm.at[s, ch, pl.ds(blk * CH_ROWS, CH_ROWS), :],
            bufs[nb], sems[nb])

      def wait(nb):
        pltpu.make_async_copy(
            img_hbm.at[0, 0, pl.ds(0, CH_ROWS), :], bufs[nb],
            sems[nb]).wait()

      def process(k, nb):
        cvec = lane_base + lax.shift_right_logical(k, 3) * BINS

        @plsc.parallel_loop(0, CH_ROWS)
        def _(r):
          for col in range(VECS_PER_ROW):
            x = bufs[nb][r, pl.ds(col * L, L)]
            f = x * jnp.float32(BINS)
            f = jnp.minimum(jnp.maximum(f, 0.0), jnp.float32(BINS - 1))
            addr = f.astype(jnp.int32) + cvec
            plsc.addupdate_scatter(table, [addr], ones)

      start(jnp.int32(0), 0)

      def outer(k2, carry):
        a = k2 * 2
        start(a + 1, 1)
        wait(0)
        process(a, 0)

        @pl.when(k2 < nchunks // 2 - 1)
        def _():
          start(a + 2, 0)

        wait(1)
        process(a + 1, 1)
        return carry

      lax.fori_loop(0, nchunks // 2, outer, 0)

    @pl.when(c == 0)
    def _():
      run(fake_hbm)

    @pl.when(c == 1)
    def _():
      run(real_hbm)

    for j in range(TBL // L):
      v = table[pl.ds(j * L, L)]
      for l in range(1, L):
        v = v + table[pl.ds(l * TBL + j * L, L)]
      acc[pl.ds(j * L, L)] = v

    pltpu.sync_copy(acc, out_hbm.at[c * NS + s])

  return hist_kernel(fake_images, real_images)


def _loss_body(h_ref, o_ref):
  h = h_ref[...]
  ssum = jnp.clip(jnp.sum(h, axis=1, keepdims=True), 1e-8, None)
  n = h / ssum
  d = jnp.abs(n[:ROWS] - n[ROWS:])
  o_ref[0, 0] = jnp.sum(d) / jnp.float32(ROWS * BINS)


def kernel(fake_images, real_images):
  hists = _histograms_sc(fake_images, real_images).reshape(NC * ROWS, BINS)
  loss = pl.pallas_call(
      _loss_body,
      out_shape=jax.ShapeDtypeStruct((1, 1), jnp.float32),
      out_specs=pl.BlockSpec(memory_space=pltpu.SMEM),
  )(hists)
  return loss[0, 0]
